# baseline (speedup 1.0000x reference)
import functools

import jax
import jax.numpy as jnp
from jax.experimental import pallas as pl
from jax.experimental.pallas import tpu as pltpu

HSV_BINS = 32
NEG_SLOPE = 0.2
H = 512
W = 512
CH = 16

_OFFSETS = ((0, 0), (0, 1), (0, 2), (1, 2), (2, 2), (2, 1), (2, 0), (1, 0))


def _shift(a, dy, dx):
    if dy == 0:
        a = jnp.concatenate([a[1:2, :], a[:-1, :]], axis=0)
    elif dy == 2:
        a = jnp.concatenate([a[1:, :], a[-2:-1, :]], axis=0)
    if dx == 0:
        a = jnp.concatenate([a[:, 1:2], a[:, :-1]], axis=1)
    elif dx == 2:
        a = jnp.concatenate([a[:, 1:], a[:, -2:-1]], axis=1)
    return a


def _img_kernel(x_ref, o_ref, code_ref, idx_ref):
    img = jnp.clip(x_ref[0], 0.0, 1.0)
    r, g, b = img[0], img[1], img[2]
    maxc = jnp.maximum(jnp.maximum(r, g), b)
    minc = jnp.minimum(jnp.minimum(r, g), b)
    delta = maxc - minc
    mask = delta > 1e-6
    safe = jnp.where(mask, delta, 1.0)
    hue = jnp.zeros_like(maxc)
    hue = jnp.where((maxc == r) & mask, jnp.mod((g - b) / safe, 6.0), hue)
    hue = jnp.where((maxc == g) & mask, (b - r) / safe + 2.0, hue)
    hue = jnp.where((maxc == b) & mask, (r - g) / safe + 4.0, hue)
    hue = hue / 6.0
    sat = jnp.where(maxc > 1e-6, delta / jnp.maximum(maxc, 1e-6),
                    jnp.zeros_like(maxc))
    chans = (hue, sat, maxc)

    for c, a in enumerate(chans):
        idx = jnp.clip(jnp.floor(a * HSV_BINS).astype(jnp.int32), 0,
                       HSV_BINS - 1)
        code = jnp.zeros(a.shape, jnp.int32)
        for bit, (dy, dx) in enumerate(_OFFSETS):
            nb = _shift(a, dy, dx)
            code = code + jnp.where(nb >= a, 1 << bit, 0)
        code_ref[c] = code
        idx_ref[c] = idx

    one = jnp.float32(1.0)
    zero = jnp.float32(0.0)

    def body(k, acc):
        s = pl.multiple_of(k * CH, CH)
        codes = [code_ref[c, pl.ds(s, CH), :] for c in range(3)]
        idxs = [idx_ref[c, pl.ds(s, CH), :] for c in range(3)]
        lhi = [x[:, :, None] >> 4 for x in codes]
        llo = [(x & 15)[:, :, None] for x in codes]
        hhi = [x[:, :, None] >> 3 for x in idxs]
        hlo = [(x & 7)[:, :, None] for x in idxs]

        li = jax.lax.broadcasted_iota(jnp.int32, (CH, W, 64), 2)
        val = jnp.where(
            li < 16, lhi[0],
            jnp.where(
                li < 32, lhi[1],
                jnp.where(
                    li < 48, lhi[2],
                    jnp.where(li < 52, hhi[0],
                              jnp.where(li < 56, hhi[1], hhi[2])))))
        tgt = jnp.where(li < 48, li & 15,
                        jnp.where(li < 60, (li - 48) & 3, -1))
        lmat = jnp.where(val == tgt, one, zero).astype(
            jnp.bfloat16).reshape(CH * W, 64)

        ri = jax.lax.broadcasted_iota(jnp.int32, (CH, W, 128), 2)
        rval = jnp.where(
            ri < 16, llo[0],
            jnp.where(
                ri < 32, llo[1],
                jnp.where(
                    ri < 48, llo[2],
                    jnp.where(ri < 56, hlo[0],
                              jnp.where(ri < 64, hlo[1], hlo[2])))))
        rtgt = jnp.where(ri < 48, ri & 15,
                         jnp.where(ri < 72, (ri - 48) & 7, -1))
        rmat = jnp.where(rval == rtgt, one, zero).astype(
            jnp.bfloat16).reshape(CH * W, 128)

        return acc + jax.lax.dot_general(
            lmat, rmat, (((0,), (0,)), ((), ())),
            preferred_element_type=jnp.float32)

    acc = jax.lax.fori_loop(0, H // CH, body,
                            jnp.zeros((64, 128), jnp.float32))
    o_ref[0] = acc * (1.0 / float(H * W))


def _head_kernel(hi_ref, lo_ref, w_ref, b_ref, env_ref, sea_ref,
                 cf_ref, cc_ref, *, batch, t):
    mh = jnp.mean(hi_ref[...].reshape(batch, t, 2048), axis=1)
    ml = jnp.mean(lo_ref[...].reshape(batch, t, 864), axis=1)
    z = (jnp.einsum('bk,ok->bo', mh, w_ref[:, :2048],
                    preferred_element_type=jnp.float32)
         + jnp.einsum('bk,ok->bo', ml, w_ref[:, 2048:],
                      preferred_element_type=jnp.float32)
         + b_ref[...])
    cf_ref[...] = jnp.where(z >= 0, z, NEG_SLOPE * z)
    j = jax.lax.broadcasted_iota(jnp.int32, (batch, 8), 1)
    enc = jnp.where(j < 4, env_ref[...], sea_ref[...] + 4)
    cc_ref[...] = jnp.where(j == enc, 1.0, 0.0)


@jax.jit
def kernel(sequence, high, W_, b, environment_id, season_id):
    B, T, C, _, _ = sequence.shape
    BT = B * T
    seq = sequence.reshape(BT, C, H, W)

    counts = pl.pallas_call(
        _img_kernel,
        grid=(BT,),
        in_specs=[pl.BlockSpec((1, C, H, W), lambda i: (i, 0, 0, 0))],
        out_specs=pl.BlockSpec((1, 64, 128), lambda i: (i, 0, 0)),
        out_shape=jax.ShapeDtypeStruct((BT, 64, 128), jnp.float32),
        scratch_shapes=[pltpu.VMEM((3, H, W), jnp.int32),
                        pltpu.VMEM((3, H, W), jnp.int32)],
        compiler_params=pltpu.CompilerParams(
            dimension_semantics=("parallel",),
            vmem_limit_bytes=63 * 1024 * 1024,
        ),
    )(seq)

    parts = []
    for c in range(3):
        parts.append(counts[:, 48 + 4 * c:52 + 4 * c,
                            48 + 8 * c:56 + 8 * c].reshape(BT, 32))
    for c in range(3):
        parts.append(counts[:, 16 * c:16 * c + 16,
                            16 * c:16 * c + 16].reshape(BT, 256))
    low = jnp.concatenate(parts, axis=1)

    c_feat, c_cls = pl.pallas_call(
        functools.partial(_head_kernel, batch=B, t=T),
        out_shape=(jax.ShapeDtypeStruct((B, 512), jnp.float32),
                   jax.ShapeDtypeStruct((B, 8), jnp.float32)),
    )(high, low, W_, b.reshape(1, 512),
      environment_id.astype(jnp.int32).reshape(B, 1),
      season_id.astype(jnp.int32).reshape(B, 1))
    return (c_feat, c_cls)


# speedup vs baseline: 15.6701x; 15.6701x over previous
import functools

import jax
import jax.numpy as jnp
from jax.experimental import pallas as pl
from jax.experimental.pallas import tpu as pltpu

HSV_BINS = 32
NEG_SLOPE = 0.2
H = 512
W = 512
CH = 8

_OFFSETS = ((0, 0), (0, 1), (0, 2), (1, 2), (2, 2), (2, 1), (2, 0), (1, 0))


def _shift(a, dy, dx):
    if dy == 0:
        a = jnp.concatenate([a[1:2, :], a[:-1, :]], axis=0)
    elif dy == 2:
        a = jnp.concatenate([a[1:, :], a[-2:-1, :]], axis=0)
    if dx == 0:
        a = jnp.concatenate([a[:, 1:2], a[:, :-1]], axis=1)
    elif dx == 2:
        a = jnp.concatenate([a[:, 1:], a[:, -2:-1]], axis=1)
    return a


def _img_kernel(x_ref, o_ref, code_ref):
    img = jnp.clip(x_ref[0], 0.0, 1.0)
    r, g, b = img[0], img[1], img[2]
    maxc = jnp.maximum(jnp.maximum(r, g), b)
    minc = jnp.minimum(jnp.minimum(r, g), b)
    delta = maxc - minc
    mask = delta > 1e-6
    safe = jnp.where(mask, delta, 1.0)
    hue = jnp.zeros_like(maxc)
    hue = jnp.where((maxc == r) & mask, jnp.mod((g - b) / safe, 6.0), hue)
    hue = jnp.where((maxc == g) & mask, (b - r) / safe + 2.0, hue)
    hue = jnp.where((maxc == b) & mask, (r - g) / safe + 4.0, hue)
    hue = hue / 6.0
    sat = jnp.where(maxc > 1e-6, delta / jnp.maximum(maxc, 1e-6),
                    jnp.zeros_like(maxc))
    chans = (hue, sat, maxc)

    for c, a in enumerate(chans):
        idx = jnp.clip(jnp.floor(a * HSV_BINS).astype(jnp.int32), 0,
                       HSV_BINS - 1)
        code = jnp.zeros(a.shape, jnp.int32)
        for bit, (dy, dx) in enumerate(_OFFSETS):
            nb = _shift(a, dy, dx)
            code = code + jnp.where(nb >= a, 1 << bit, 0)
        code_ref[c] = code + (idx << 8)

    one = jnp.float32(1.0)
    zero = jnp.float32(0.0)
    si16 = jax.lax.broadcasted_iota(jnp.int32, (16, W), 0)
    si32 = jax.lax.broadcasted_iota(jnp.int32, (32, W), 0)

    def body(k, acc):
        s = pl.multiple_of(k * CH, CH)
        pk = code_ref[:, pl.ds(s, CH), :]
        lhi = (pk >> 4) & 15
        llo = pk & 15
        hhi = (pk >> 11) & 3
        hlo = (pk >> 8) & 7
        lrows, rrows = [], []
        for r in range(CH):
            lparts = [
                jnp.where(
                    jnp.broadcast_to(lhi[c, r:r + 1, :], (16, W)) == si16,
                    one, zero) for c in range(3)
            ]
            vh = jnp.where(
                si16 < 4, jnp.broadcast_to(hhi[0, r:r + 1, :], (16, W)),
                jnp.where(si16 < 8,
                          jnp.broadcast_to(hhi[1, r:r + 1, :], (16, W)),
                          jnp.broadcast_to(hhi[2, r:r + 1, :], (16, W))))
            lparts.append(jnp.where(vh == (si16 & 3), one, zero))
            lrows.append(jnp.concatenate(lparts, axis=0))

            rparts = [
                jnp.where(
                    jnp.broadcast_to(llo[c, r:r + 1, :], (16, W)) == si16,
                    one, zero) for c in range(3)
            ]
            vl = jnp.where(
                si32 < 8, jnp.broadcast_to(hlo[0, r:r + 1, :], (32, W)),
                jnp.where(si32 < 16,
                          jnp.broadcast_to(hlo[1, r:r + 1, :], (32, W)),
                          jnp.broadcast_to(hlo[2, r:r + 1, :], (32, W))))
            rparts.append(jnp.where(vl == (si32 & 7), one, zero))
            rrows.append(jnp.concatenate(rparts, axis=0))

        lmat = jnp.concatenate(lrows, axis=1).astype(jnp.bfloat16)
        rmat = jnp.concatenate(rrows, axis=1).astype(jnp.bfloat16)
        return acc + jax.lax.dot_general(
            lmat, rmat, (((1,), (1,)), ((), ())),
            preferred_element_type=jnp.float32)

    acc = jax.lax.fori_loop(0, H // CH, body,
                            jnp.zeros((64, 80), jnp.float32))
    o_ref[0] = acc * (1.0 / float(H * W))


def _head_kernel(hi_ref, lo_ref, w_ref, b_ref, env_ref, sea_ref,
                 cf_ref, cc_ref, *, batch, t):
    mh = jnp.mean(hi_ref[...].reshape(batch, t, 2048), axis=1)
    ml = jnp.mean(lo_ref[...].reshape(batch, t, 864), axis=1)
    z = (jnp.einsum('bk,ok->bo', mh, w_ref[:, :2048],
                    preferred_element_type=jnp.float32)
         + jnp.einsum('bk,ok->bo', ml, w_ref[:, 2048:],
                      preferred_element_type=jnp.float32)
         + b_ref[...])
    cf_ref[...] = jnp.where(z >= 0, z, NEG_SLOPE * z)
    j = jax.lax.broadcasted_iota(jnp.int32, (batch, 8), 1)
    enc = jnp.where(j < 4, env_ref[...], sea_ref[...] + 4)
    cc_ref[...] = jnp.where(j == enc, 1.0, 0.0)


@jax.jit
def kernel(sequence, high, W_, b, environment_id, season_id):
    B, T, C, _, _ = sequence.shape
    BT = B * T
    seq = sequence.reshape(BT, C, H, W)

    counts = pl.pallas_call(
        _img_kernel,
        grid=(BT,),
        in_specs=[pl.BlockSpec((1, C, H, W), lambda i: (i, 0, 0, 0))],
        out_specs=pl.BlockSpec((1, 64, 80), lambda i: (i, 0, 0)),
        out_shape=jax.ShapeDtypeStruct((BT, 64, 80), jnp.float32),
        scratch_shapes=[pltpu.VMEM((3, H, W), jnp.int32)],
        compiler_params=pltpu.CompilerParams(
            dimension_semantics=("parallel",),
            vmem_limit_bytes=63 * 1024 * 1024,
        ),
    )(seq)

    parts = []
    for c in range(3):
        parts.append(counts[:, 48 + 4 * c:52 + 4 * c,
                            48 + 8 * c:56 + 8 * c].reshape(BT, 32))
    for c in range(3):
        parts.append(counts[:, 16 * c:16 * c + 16,
                            16 * c:16 * c + 16].reshape(BT, 256))
    low = jnp.concatenate(parts, axis=1)

    c_feat, c_cls = pl.pallas_call(
        functools.partial(_head_kernel, batch=B, t=T),
        out_shape=(jax.ShapeDtypeStruct((B, 512), jnp.float32),
                   jax.ShapeDtypeStruct((B, 8), jnp.float32)),
    )(high, low, W_, b.reshape(1, 512),
      environment_id.astype(jnp.int32).reshape(B, 1),
      season_id.astype(jnp.int32).reshape(B, 1))
    return (c_feat, c_cls)


# speedup vs baseline: 15.6913x; 1.0013x over previous
import functools

import jax
import jax.numpy as jnp
from jax.experimental import pallas as pl
from jax.experimental.pallas import tpu as pltpu

HSV_BINS = 32
NEG_SLOPE = 0.2
H = 512
W = 512
CH = 8

_OFFSETS = ((0, 0), (0, 1), (0, 2), (1, 2), (2, 2), (2, 1), (2, 0), (1, 0))


def _shift(a, dy, dx):
    if dy == 0:
        a = jnp.concatenate([a[1:2, :], a[:-1, :]], axis=0)
    elif dy == 2:
        a = jnp.concatenate([a[1:, :], a[-2:-1, :]], axis=0)
    if dx == 0:
        a = jnp.concatenate([a[:, 1:2], a[:, :-1]], axis=1)
    elif dx == 2:
        a = jnp.concatenate([a[:, 1:], a[:, -2:-1]], axis=1)
    return a


def _img_kernel(x_ref, o_ref, code_ref):
    img = jnp.clip(x_ref[0], 0.0, 1.0)
    r, g, b = img[0], img[1], img[2]
    maxc = jnp.maximum(jnp.maximum(r, g), b)
    minc = jnp.minimum(jnp.minimum(r, g), b)
    delta = maxc - minc
    mask = delta > 1e-6
    safe = jnp.where(mask, delta, 1.0)
    hue = jnp.zeros_like(maxc)
    hue = jnp.where((maxc == r) & mask, jnp.mod((g - b) / safe, 6.0), hue)
    hue = jnp.where((maxc == g) & mask, (b - r) / safe + 2.0, hue)
    hue = jnp.where((maxc == b) & mask, (r - g) / safe + 4.0, hue)
    hue = hue / 6.0
    sat = jnp.where(maxc > 1e-6, delta / jnp.maximum(maxc, 1e-6),
                    jnp.zeros_like(maxc))
    chans = (hue, sat, maxc)

    for c, a in enumerate(chans):
        idx = jnp.clip(jnp.floor(a * HSV_BINS).astype(jnp.int32), 0,
                       HSV_BINS - 1)
        code = jnp.zeros(a.shape, jnp.int32)
        for bit, (dy, dx) in enumerate(_OFFSETS):
            nb = _shift(a, dy, dx)
            code = code + jnp.where(nb >= a, 1 << bit, 0)
        code_ref[c] = code + (idx << 8)

    one = jnp.bfloat16(1.0)
    zero = jnp.bfloat16(0.0)
    si16i = jax.lax.broadcasted_iota(jnp.int32, (16, W), 0)
    si32i = jax.lax.broadcasted_iota(jnp.int32, (32, W), 0)
    si16 = si16i.astype(jnp.bfloat16)
    t16 = (si16i & 3).astype(jnp.bfloat16)
    si32 = si32i.astype(jnp.bfloat16)
    t32 = (si32i & 7).astype(jnp.bfloat16)
    bf4 = jnp.bfloat16(4.0)
    bf8 = jnp.bfloat16(8.0)
    bf16_ = jnp.bfloat16(16.0)

    def body(k, acc):
        s = pl.multiple_of(k * CH, CH)
        pk = code_ref[:, pl.ds(s, CH), :]
        lhi = ((pk >> 4) & 15).astype(jnp.bfloat16)
        llo = (pk & 15).astype(jnp.bfloat16)
        hhi = ((pk >> 11) & 3).astype(jnp.bfloat16)
        hlo = ((pk >> 8) & 7).astype(jnp.bfloat16)
        lrows, rrows = [], []
        for r in range(CH):
            lparts = [
                jnp.where(
                    jnp.broadcast_to(lhi[c, r:r + 1, :], (16, W)) == si16,
                    one, zero) for c in range(3)
            ]
            vh = jnp.where(
                si16 < bf4, jnp.broadcast_to(hhi[0, r:r + 1, :], (16, W)),
                jnp.where(si16 < bf8,
                          jnp.broadcast_to(hhi[1, r:r + 1, :], (16, W)),
                          jnp.broadcast_to(hhi[2, r:r + 1, :], (16, W))))
            lparts.append(jnp.where(vh == t16, one, zero))
            lrows.append(jnp.concatenate(lparts, axis=0))

            rparts = [
                jnp.where(
                    jnp.broadcast_to(llo[c, r:r + 1, :], (16, W)) == si16,
                    one, zero) for c in range(3)
            ]
            vl = jnp.where(
                si32 < bf8, jnp.broadcast_to(hlo[0, r:r + 1, :], (32, W)),
                jnp.where(si32 < bf16_,
                          jnp.broadcast_to(hlo[1, r:r + 1, :], (32, W)),
                          jnp.broadcast_to(hlo[2, r:r + 1, :], (32, W))))
            rparts.append(jnp.where(vl == t32, one, zero))
            rrows.append(jnp.concatenate(rparts, axis=0))

        lmat = jnp.concatenate(lrows, axis=1)
        rmat = jnp.concatenate(rrows, axis=1)
        return acc + jax.lax.dot_general(
            lmat, rmat, (((1,), (1,)), ((), ())),
            preferred_element_type=jnp.float32)

    acc = jax.lax.fori_loop(0, H // CH, body,
                            jnp.zeros((64, 80), jnp.float32))
    o_ref[0] = acc * (1.0 / float(H * W))


def _head_kernel(hi_ref, lo_ref, w_ref, b_ref, env_ref, sea_ref,
                 cf_ref, cc_ref, *, batch, t):
    mh = jnp.mean(hi_ref[...].reshape(batch, t, 2048), axis=1)
    ml = jnp.mean(lo_ref[...].reshape(batch, t, 864), axis=1)
    z = (jnp.einsum('bk,ok->bo', mh, w_ref[:, :2048],
                    preferred_element_type=jnp.float32)
         + jnp.einsum('bk,ok->bo', ml, w_ref[:, 2048:],
                      preferred_element_type=jnp.float32)
         + b_ref[...])
    cf_ref[...] = jnp.where(z >= 0, z, NEG_SLOPE * z)
    j = jax.lax.broadcasted_iota(jnp.int32, (batch, 8), 1)
    enc = jnp.where(j < 4, env_ref[...], sea_ref[...] + 4)
    cc_ref[...] = jnp.where(j == enc, 1.0, 0.0)


@jax.jit
def kernel(sequence, high, W_, b, environment_id, season_id):
    B, T, C, _, _ = sequence.shape
    BT = B * T
    seq = sequence.reshape(BT, C, H, W)

    counts = pl.pallas_call(
        _img_kernel,
        grid=(BT,),
        in_specs=[pl.BlockSpec((1, C, H, W), lambda i: (i, 0, 0, 0))],
        out_specs=pl.BlockSpec((1, 64, 80), lambda i: (i, 0, 0)),
        out_shape=jax.ShapeDtypeStruct((BT, 64, 80), jnp.float32),
        scratch_shapes=[pltpu.VMEM((3, H, W), jnp.int32)],
        compiler_params=pltpu.CompilerParams(
            dimension_semantics=("parallel",),
            vmem_limit_bytes=63 * 1024 * 1024,
        ),
    )(seq)

    parts = []
    for c in range(3):
        parts.append(counts[:, 48 + 4 * c:52 + 4 * c,
                            48 + 8 * c:56 + 8 * c].reshape(BT, 32))
    for c in range(3):
        parts.append(counts[:, 16 * c:16 * c + 16,
                            16 * c:16 * c + 16].reshape(BT, 256))
    low = jnp.concatenate(parts, axis=1)

    c_feat, c_cls = pl.pallas_call(
        functools.partial(_head_kernel, batch=B, t=T),
        out_shape=(jax.ShapeDtypeStruct((B, 512), jnp.float32),
                   jax.ShapeDtypeStruct((B, 8), jnp.float32)),
    )(high, low, W_, b.reshape(1, 512),
      environment_id.astype(jnp.int32).reshape(B, 1),
      season_id.astype(jnp.int32).reshape(B, 1))
    return (c_feat, c_cls)


# speedup vs baseline: 18.6362x; 1.1877x over previous
import functools

import jax
import jax.numpy as jnp
from jax.experimental import pallas as pl
from jax.experimental.pallas import tpu as pltpu

HSV_BINS = 32
NEG_SLOPE = 0.2
H = 512
W = 512
CH = 16

_OFFSETS = ((0, 0), (0, 1), (0, 2), (1, 2), (2, 2), (2, 1), (2, 0), (1, 0))


def _shift(a, dy, dx):
    if dy == 0:
        a = jnp.concatenate([a[1:2, :], a[:-1, :]], axis=0)
    elif dy == 2:
        a = jnp.concatenate([a[1:, :], a[-2:-1, :]], axis=0)
    if dx == 0:
        a = jnp.concatenate([a[:, 1:2], a[:, :-1]], axis=1)
    elif dx == 2:
        a = jnp.concatenate([a[:, 1:], a[:, -2:-1]], axis=1)
    return a


def _img_kernel(x_ref, o_ref, code_ref):
    img = jnp.clip(x_ref[0], 0.0, 1.0)
    r, g, b = img[0], img[1], img[2]
    maxc = jnp.maximum(jnp.maximum(r, g), b)
    minc = jnp.minimum(jnp.minimum(r, g), b)
    delta = maxc - minc
    mask = delta > 1e-6
    safe = jnp.where(mask, delta, 1.0)
    hue = jnp.zeros_like(maxc)
    hue = jnp.where((maxc == r) & mask, jnp.mod((g - b) / safe, 6.0), hue)
    hue = jnp.where((maxc == g) & mask, (b - r) / safe + 2.0, hue)
    hue = jnp.where((maxc == b) & mask, (r - g) / safe + 4.0, hue)
    hue = hue / 6.0
    sat = jnp.where(maxc > 1e-6, delta / jnp.maximum(maxc, 1e-6),
                    jnp.zeros_like(maxc))
    chans = (hue, sat, maxc)

    for c, a in enumerate(chans):
        idx = jnp.clip(jnp.floor(a * HSV_BINS).astype(jnp.int32), 0,
                       HSV_BINS - 1)
        code = jnp.zeros(a.shape, jnp.int32)
        for bit, (dy, dx) in enumerate(_OFFSETS):
            nb = _shift(a, dy, dx)
            code = code + jnp.where(nb >= a, 1 << bit, 0)
        code_ref[c] = code + (idx << 8)

    one = jnp.bfloat16(1.0)
    zero = jnp.bfloat16(0.0)
    si16i = jax.lax.broadcasted_iota(jnp.int32, (16, W), 0)
    si32i = jax.lax.broadcasted_iota(jnp.int32, (32, W), 0)
    si16 = si16i.astype(jnp.bfloat16)
    t16 = (si16i & 3).astype(jnp.bfloat16)
    si32 = si32i.astype(jnp.bfloat16)
    t32 = (si32i & 7).astype(jnp.bfloat16)
    bf4 = jnp.bfloat16(4.0)
    bf8 = jnp.bfloat16(8.0)
    bf16_ = jnp.bfloat16(16.0)

    def body(k, acc):
        s = pl.multiple_of(k * CH, CH)
        pk = code_ref[:, pl.ds(s, CH), :]
        lhi = ((pk >> 4) & 15).astype(jnp.bfloat16)
        llo = (pk & 15).astype(jnp.bfloat16)
        hhi = ((pk >> 11) & 3).astype(jnp.bfloat16)
        hlo = ((pk >> 8) & 7).astype(jnp.bfloat16)
        lrows, rrows = [], []
        for r in range(CH):
            lparts = [
                jnp.where(
                    jnp.broadcast_to(lhi[c, r:r + 1, :], (16, W)) == si16,
                    one, zero) for c in range(3)
            ]
            vh = jnp.where(
                si16 < bf4, jnp.broadcast_to(hhi[0, r:r + 1, :], (16, W)),
                jnp.where(si16 < bf8,
                          jnp.broadcast_to(hhi[1, r:r + 1, :], (16, W)),
                          jnp.broadcast_to(hhi[2, r:r + 1, :], (16, W))))
            lparts.append(jnp.where(vh == t16, one, zero))
            lrows.append(jnp.concatenate(lparts, axis=0))

            rparts = [
                jnp.where(
                    jnp.broadcast_to(llo[c, r:r + 1, :], (16, W)) == si16,
                    one, zero) for c in range(3)
            ]
            vl = jnp.where(
                si32 < bf8, jnp.broadcast_to(hlo[0, r:r + 1, :], (32, W)),
                jnp.where(si32 < bf16_,
                          jnp.broadcast_to(hlo[1, r:r + 1, :], (32, W)),
                          jnp.broadcast_to(hlo[2, r:r + 1, :], (32, W))))
            rparts.append(jnp.where(vl == t32, one, zero))
            rrows.append(jnp.concatenate(rparts, axis=0))

        lmat = jnp.concatenate(lrows, axis=1)
        rmat = jnp.concatenate(rrows, axis=1)
        return acc + jax.lax.dot_general(
            lmat, rmat, (((1,), (1,)), ((), ())),
            preferred_element_type=jnp.float32)

    acc = jax.lax.fori_loop(0, H // CH, body,
                            jnp.zeros((64, 80), jnp.float32))
    o_ref[0] = acc * (1.0 / float(H * W))


def _head_kernel(hi_ref, lo_ref, w_ref, b_ref, env_ref, sea_ref,
                 cf_ref, cc_ref, *, batch, t):
    mh = jnp.mean(hi_ref[...].reshape(batch, t, 2048), axis=1)
    ml = jnp.mean(lo_ref[...].reshape(batch, t, 864), axis=1)
    z = (jnp.einsum('bk,ok->bo', mh, w_ref[:, :2048],
                    preferred_element_type=jnp.float32)
         + jnp.einsum('bk,ok->bo', ml, w_ref[:, 2048:],
                      preferred_element_type=jnp.float32)
         + b_ref[...])
    cf_ref[...] = jnp.where(z >= 0, z, NEG_SLOPE * z)
    j = jax.lax.broadcasted_iota(jnp.int32, (batch, 8), 1)
    enc = jnp.where(j < 4, env_ref[...], sea_ref[...] + 4)
    cc_ref[...] = jnp.where(j == enc, 1.0, 0.0)


@jax.jit
def kernel(sequence, high, W_, b, environment_id, season_id):
    B, T, C, _, _ = sequence.shape
    BT = B * T
    seq = sequence.reshape(BT, C, H, W)

    counts = pl.pallas_call(
        _img_kernel,
        grid=(BT,),
        in_specs=[pl.BlockSpec((1, C, H, W), lambda i: (i, 0, 0, 0))],
        out_specs=pl.BlockSpec((1, 64, 80), lambda i: (i, 0, 0)),
        out_shape=jax.ShapeDtypeStruct((BT, 64, 80), jnp.float32),
        scratch_shapes=[pltpu.VMEM((3, H, W), jnp.int32)],
        compiler_params=pltpu.CompilerParams(
            dimension_semantics=("parallel",),
            vmem_limit_bytes=63 * 1024 * 1024,
        ),
    )(seq)

    parts = []
    for c in range(3):
        parts.append(counts[:, 48 + 4 * c:52 + 4 * c,
                            48 + 8 * c:56 + 8 * c].reshape(BT, 32))
    for c in range(3):
        parts.append(counts[:, 16 * c:16 * c + 16,
                            16 * c:16 * c + 16].reshape(BT, 256))
    low = jnp.concatenate(parts, axis=1)

    c_feat, c_cls = pl.pallas_call(
        functools.partial(_head_kernel, batch=B, t=T),
        out_shape=(jax.ShapeDtypeStruct((B, 512), jnp.float32),
                   jax.ShapeDtypeStruct((B, 8), jnp.float32)),
    )(high, low, W_, b.reshape(1, 512),
      environment_id.astype(jnp.int32).reshape(B, 1),
      season_id.astype(jnp.int32).reshape(B, 1))
    return (c_feat, c_cls)


# speedup vs baseline: 19.9507x; 1.0705x over previous
import functools

import jax
import jax.numpy as jnp
from jax.experimental import pallas as pl
from jax.experimental.pallas import tpu as pltpu

HSV_BINS = 32
NEG_SLOPE = 0.2
H = 512
W = 512
CH = 32

_OFFSETS = ((0, 0), (0, 1), (0, 2), (1, 2), (2, 2), (2, 1), (2, 0), (1, 0))


def _shift(a, dy, dx):
    if dy == 0:
        a = jnp.concatenate([a[1:2, :], a[:-1, :]], axis=0)
    elif dy == 2:
        a = jnp.concatenate([a[1:, :], a[-2:-1, :]], axis=0)
    if dx == 0:
        a = jnp.concatenate([a[:, 1:2], a[:, :-1]], axis=1)
    elif dx == 2:
        a = jnp.concatenate([a[:, 1:], a[:, -2:-1]], axis=1)
    return a


def _img_kernel(x_ref, o_ref, code_ref):
    img = jnp.clip(x_ref[0], 0.0, 1.0)
    r, g, b = img[0], img[1], img[2]
    maxc = jnp.maximum(jnp.maximum(r, g), b)
    minc = jnp.minimum(jnp.minimum(r, g), b)
    delta = maxc - minc
    mask = delta > 1e-6
    safe = jnp.where(mask, delta, 1.0)
    hue = jnp.zeros_like(maxc)
    hue = jnp.where((maxc == r) & mask, jnp.mod((g - b) / safe, 6.0), hue)
    hue = jnp.where((maxc == g) & mask, (b - r) / safe + 2.0, hue)
    hue = jnp.where((maxc == b) & mask, (r - g) / safe + 4.0, hue)
    hue = hue / 6.0
    sat = jnp.where(maxc > 1e-6, delta / jnp.maximum(maxc, 1e-6),
                    jnp.zeros_like(maxc))
    chans = (hue, sat, maxc)

    for c, a in enumerate(chans):
        idx = jnp.clip(jnp.floor(a * HSV_BINS).astype(jnp.int32), 0,
                       HSV_BINS - 1)
        code = jnp.zeros(a.shape, jnp.int32)
        for bit, (dy, dx) in enumerate(_OFFSETS):
            nb = _shift(a, dy, dx)
            code = code + jnp.where(nb >= a, 1 << bit, 0)
        code_ref[c] = code + (idx << 8)

    one = jnp.bfloat16(1.0)
    zero = jnp.bfloat16(0.0)
    si16i = jax.lax.broadcasted_iota(jnp.int32, (16, W), 0)
    si32i = jax.lax.broadcasted_iota(jnp.int32, (32, W), 0)
    si16 = si16i.astype(jnp.bfloat16)
    t16 = (si16i & 3).astype(jnp.bfloat16)
    si32 = si32i.astype(jnp.bfloat16)
    t32 = (si32i & 7).astype(jnp.bfloat16)
    bf4 = jnp.bfloat16(4.0)
    bf8 = jnp.bfloat16(8.0)
    bf16_ = jnp.bfloat16(16.0)

    def body(k, acc):
        s = pl.multiple_of(k * CH, CH)
        pk = code_ref[:, pl.ds(s, CH), :]
        lhi = ((pk >> 4) & 15).astype(jnp.bfloat16)
        llo = (pk & 15).astype(jnp.bfloat16)
        hhi = ((pk >> 11) & 3).astype(jnp.bfloat16)
        hlo = ((pk >> 8) & 7).astype(jnp.bfloat16)
        lrows, rrows = [], []
        for r in range(CH):
            lparts = [
                jnp.where(
                    jnp.broadcast_to(lhi[c, r:r + 1, :], (16, W)) == si16,
                    one, zero) for c in range(3)
            ]
            vh = jnp.where(
                si16 < bf4, jnp.broadcast_to(hhi[0, r:r + 1, :], (16, W)),
                jnp.where(si16 < bf8,
                          jnp.broadcast_to(hhi[1, r:r + 1, :], (16, W)),
                          jnp.broadcast_to(hhi[2, r:r + 1, :], (16, W))))
            lparts.append(jnp.where(vh == t16, one, zero))
            lrows.append(jnp.concatenate(lparts, axis=0))

            rparts = [
                jnp.where(
                    jnp.broadcast_to(llo[c, r:r + 1, :], (16, W)) == si16,
                    one, zero) for c in range(3)
            ]
            vl = jnp.where(
                si32 < bf8, jnp.broadcast_to(hlo[0, r:r + 1, :], (32, W)),
                jnp.where(si32 < bf16_,
                          jnp.broadcast_to(hlo[1, r:r + 1, :], (32, W)),
                          jnp.broadcast_to(hlo[2, r:r + 1, :], (32, W))))
            rparts.append(jnp.where(vl == t32, one, zero))
            rrows.append(jnp.concatenate(rparts, axis=0))

        lmat = jnp.concatenate(lrows, axis=1)
        rmat = jnp.concatenate(rrows, axis=1)
        return acc + jax.lax.dot_general(
            lmat, rmat, (((1,), (1,)), ((), ())),
            preferred_element_type=jnp.float32)

    acc = jax.lax.fori_loop(0, H // CH, body,
                            jnp.zeros((64, 80), jnp.float32))
    o_ref[0] = acc * (1.0 / float(H * W))


def _head_kernel(hi_ref, lo_ref, w_ref, b_ref, env_ref, sea_ref,
                 cf_ref, cc_ref, *, batch, t):
    mh = jnp.mean(hi_ref[...].reshape(batch, t, 2048), axis=1)
    ml = jnp.mean(lo_ref[...].reshape(batch, t, 864), axis=1)
    z = (jnp.einsum('bk,ok->bo', mh, w_ref[:, :2048],
                    preferred_element_type=jnp.float32)
         + jnp.einsum('bk,ok->bo', ml, w_ref[:, 2048:],
                      preferred_element_type=jnp.float32)
         + b_ref[...])
    cf_ref[...] = jnp.where(z >= 0, z, NEG_SLOPE * z)
    j = jax.lax.broadcasted_iota(jnp.int32, (batch, 8), 1)
    enc = jnp.where(j < 4, env_ref[...], sea_ref[...] + 4)
    cc_ref[...] = jnp.where(j == enc, 1.0, 0.0)


@jax.jit
def kernel(sequence, high, W_, b, environment_id, season_id):
    B, T, C, _, _ = sequence.shape
    BT = B * T
    seq = sequence.reshape(BT, C, H, W)

    counts = pl.pallas_call(
        _img_kernel,
        grid=(BT,),
        in_specs=[pl.BlockSpec((1, C, H, W), lambda i: (i, 0, 0, 0))],
        out_specs=pl.BlockSpec((1, 64, 80), lambda i: (i, 0, 0)),
        out_shape=jax.ShapeDtypeStruct((BT, 64, 80), jnp.float32),
        scratch_shapes=[pltpu.VMEM((3, H, W), jnp.int32)],
        compiler_params=pltpu.CompilerParams(
            dimension_semantics=("parallel",),
            vmem_limit_bytes=63 * 1024 * 1024,
        ),
    )(seq)

    parts = []
    for c in range(3):
        parts.append(counts[:, 48 + 4 * c:52 + 4 * c,
                            48 + 8 * c:56 + 8 * c].reshape(BT, 32))
    for c in range(3):
        parts.append(counts[:, 16 * c:16 * c + 16,
                            16 * c:16 * c + 16].reshape(BT, 256))
    low = jnp.concatenate(parts, axis=1)

    c_feat, c_cls = pl.pallas_call(
        functools.partial(_head_kernel, batch=B, t=T),
        out_shape=(jax.ShapeDtypeStruct((B, 512), jnp.float32),
                   jax.ShapeDtypeStruct((B, 8), jnp.float32)),
    )(high, low, W_, b.reshape(1, 512),
      environment_id.astype(jnp.int32).reshape(B, 1),
      season_id.astype(jnp.int32).reshape(B, 1))
    return (c_feat, c_cls)
